# baseline (speedup 1.0000x reference)
import functools

import jax
import jax.numpy as jnp
from jax import lax
from jax.experimental import pallas as pl
from jax.experimental.pallas import tpu as pltpu
from jax.experimental.pallas import tpu_sc as plsc

EMB_DIM = 64
EMB_NUM = 12
NQUAD = EMB_NUM ** 4
QD = 4 * EMB_DIM

NC = 2
NS = 16
NW = NC * NS

CHUNK = 128
NBUF = 2


def _prep_body(pe_ref, gamma_ref, beta_ref, w_ref, b_ref, x_ref,
               tab4_ref, qidx_ref):
    n = EMB_NUM
    row = lax.broadcasted_iota(jnp.int32, (n, n), 0)
    col = lax.broadcasted_iota(jnp.int32, (n, n), 1)
    h = jnp.where(row == col, jnp.float32(1.0), jnp.float32(0.0)) + jnp.float32(0.05)
    mean = jnp.mean(h, axis=1, keepdims=True)
    var = jnp.mean((h - mean) ** 2, axis=1, keepdims=True)
    hn = (h - mean) / jnp.sqrt(var + jnp.float32(1e-5))
    hn = hn * gamma_ref[...] + beta_ref[...]
    t = lax.dot_general(hn, w_ref[...], (((1,), (1,)), ((), ())),
                        preferred_element_type=jnp.float32)
    t = jnp.maximum(t + b_ref[...], jnp.float32(0.0)) + pe_ref[...]

    q_iota = lax.broadcasted_iota(jnp.int32, (NQUAD, n), 0)
    qc_iota = lax.broadcasted_iota(jnp.int32, (NQUAD, n), 1)
    for k, div in enumerate((n ** 3, n ** 2, n, 1)):
        sel = ((q_iota // div) % n == qc_iota).astype(jnp.float32)
        tab4_ref[:, k * EMB_DIM:(k + 1) * EMB_DIM] = lax.dot_general(
            sel, t, (((1,), (0,)), ((), ())), preferred_element_type=jnp.float32)

    seq_len = x_ref.shape[1]
    l_iota = lax.broadcasted_iota(jnp.int32, (seq_len, seq_len // 4), 0)
    g_iota = lax.broadcasted_iota(jnp.int32, (seq_len, seq_len // 4), 1)
    m = l_iota % 4
    pw = jnp.where(m == 0, jnp.float32(n ** 3),
                   jnp.where(m == 1, jnp.float32(n ** 2),
                             jnp.where(m == 2, jnp.float32(n), jnp.float32(1.0))))
    sq = jnp.where(l_iota // 4 == g_iota, pw, jnp.float32(0.0))
    qf = lax.dot_general(x_ref[...].astype(jnp.float32), sq,
                         (((1,), (0,)), ((), ())),
                         preferred_element_type=jnp.float32)
    qidx_ref[...] = qf.astype(jnp.int32)


def _prep(pe, gamma, beta, W, b, x):
    return pl.pallas_call(
        _prep_body,
        out_shape=[
            jax.ShapeDtypeStruct((NQUAD, QD), jnp.float32),
            jax.ShapeDtypeStruct((x.shape[0], x.shape[1] // 4), jnp.int32),
        ],
    )(pe, gamma.reshape(1, EMB_NUM), beta.reshape(1, EMB_NUM),
      W, b.reshape(1, EMB_DIM), x)


def _make_gather(total_quads):
    assert total_quads % (NW * CHUNK * NBUF) == 0
    per_w = total_quads // NW
    n_groups = per_w // (CHUNK * NBUF)
    mesh = plsc.VectorSubcoreMesh(core_axis_name="c", subcore_axis_name="s")

    @functools.partial(
        pl.kernel,
        mesh=mesh,
        compiler_params=pltpu.CompilerParams(use_tc_tiling_on_sc=True),
        out_type=jax.ShapeDtypeStruct((total_quads, QD), jnp.float32),
        scratch_types=[
            pltpu.VMEM((per_w,), jnp.int32),
            [pltpu.VMEM((CHUNK, QD), jnp.float32) for _ in range(NBUF)],
            [pltpu.SemaphoreType.DMA for _ in range(NBUF)],
            [pltpu.SemaphoreType.DMA for _ in range(NBUF)],
        ],
    )
    def gather_kernel(table_hbm, idx_hbm, out_hbm, idx_v, rows, gsems, ssems):
        sid = lax.axis_index("s")
        wid = sid * NC + lax.axis_index("c")
        base = wid * per_w
        pltpu.sync_copy(idx_hbm.at[pl.ds(base, per_w)], idx_v)

        def group(g, carry):
            for bf in range(NBUF):
                off = (g * NBUF + bf) * CHUNK

                @pl.when(g > 0)
                def _wait_prev_store():
                    pltpu.make_async_copy(
                        rows[bf], out_hbm.at[pl.ds(base + off, CHUNK)],
                        ssems[bf]).wait()

                pltpu.async_copy(
                    table_hbm.at[idx_v.at[pl.ds(off, CHUNK)]],
                    rows[bf], gsems[bf])
            for bf in range(NBUF):
                off = (g * NBUF + bf) * CHUNK
                pltpu.make_async_copy(
                    table_hbm.at[idx_v.at[pl.ds(off, CHUNK)]],
                    rows[bf], gsems[bf]).wait()
                pltpu.async_copy(rows[bf], out_hbm.at[pl.ds(base + off, CHUNK)],
                                 ssems[bf])
            return carry

        lax.fori_loop(0, n_groups, group, 0)
        for bf in range(NBUF):
            pltpu.make_async_copy(
                rows[bf], out_hbm.at[pl.ds(base, CHUNK)], ssems[bf]).wait()

    return gather_kernel


def kernel(x, pe, gamma, beta, W, b):
    Bb, Ll = x.shape
    total_quads = (Bb * Ll) // 4
    table4, qidx = _prep(pe, gamma, beta, W, b, x.astype(jnp.int32))
    out = _make_gather(total_quads)(table4, qidx.reshape(total_quads))
    return out.reshape(Bb, Ll, EMB_DIM)


# speedup vs baseline: 1.0075x; 1.0075x over previous
import functools

import jax
import jax.numpy as jnp
from jax import lax
from jax.experimental import pallas as pl
from jax.experimental.pallas import tpu as pltpu
from jax.experimental.pallas import tpu_sc as plsc

EMB_DIM = 64
EMB_NUM = 12
NQUAD = EMB_NUM ** 4
QD = 4 * EMB_DIM

NC = 2
NS = 16
NW = NC * NS

CHUNK = 64
NBUF = 4


def _prep_body(pe_ref, gamma_ref, beta_ref, w_ref, b_ref, x_ref,
               tab4_ref, qidx_ref):
    n = EMB_NUM
    row = lax.broadcasted_iota(jnp.int32, (n, n), 0)
    col = lax.broadcasted_iota(jnp.int32, (n, n), 1)
    h = jnp.where(row == col, jnp.float32(1.0), jnp.float32(0.0)) + jnp.float32(0.05)
    mean = jnp.mean(h, axis=1, keepdims=True)
    var = jnp.mean((h - mean) ** 2, axis=1, keepdims=True)
    hn = (h - mean) / jnp.sqrt(var + jnp.float32(1e-5))
    hn = hn * gamma_ref[...] + beta_ref[...]
    t = lax.dot_general(hn, w_ref[...], (((1,), (1,)), ((), ())),
                        preferred_element_type=jnp.float32)
    t = jnp.maximum(t + b_ref[...], jnp.float32(0.0)) + pe_ref[...]

    q_iota = lax.broadcasted_iota(jnp.int32, (NQUAD, n), 0)
    qc_iota = lax.broadcasted_iota(jnp.int32, (NQUAD, n), 1)
    for k, div in enumerate((n ** 3, n ** 2, n, 1)):
        sel = ((q_iota // div) % n == qc_iota).astype(jnp.float32)
        tab4_ref[:, k * EMB_DIM:(k + 1) * EMB_DIM] = lax.dot_general(
            sel, t, (((1,), (0,)), ((), ())), preferred_element_type=jnp.float32)

    seq_len = x_ref.shape[1]
    l_iota = lax.broadcasted_iota(jnp.int32, (seq_len, seq_len // 4), 0)
    g_iota = lax.broadcasted_iota(jnp.int32, (seq_len, seq_len // 4), 1)
    m = l_iota % 4
    pw = jnp.where(m == 0, jnp.float32(n ** 3),
                   jnp.where(m == 1, jnp.float32(n ** 2),
                             jnp.where(m == 2, jnp.float32(n), jnp.float32(1.0))))
    sq = jnp.where(l_iota // 4 == g_iota, pw, jnp.float32(0.0))
    qf = lax.dot_general(x_ref[...].astype(jnp.float32), sq,
                         (((1,), (0,)), ((), ())),
                         preferred_element_type=jnp.float32)
    qidx_ref[...] = qf.astype(jnp.int32)


def _prep(pe, gamma, beta, W, b, x):
    return pl.pallas_call(
        _prep_body,
        out_shape=[
            jax.ShapeDtypeStruct((NQUAD, QD), jnp.float32),
            jax.ShapeDtypeStruct((x.shape[0], x.shape[1] // 4), jnp.int32),
        ],
    )(pe, gamma.reshape(1, EMB_NUM), beta.reshape(1, EMB_NUM),
      W, b.reshape(1, EMB_DIM), x)


def _make_gather(total_quads):
    assert total_quads % (NW * CHUNK * NBUF) == 0
    per_w = total_quads // NW
    n_groups = per_w // (CHUNK * NBUF)
    mesh = plsc.VectorSubcoreMesh(core_axis_name="c", subcore_axis_name="s")

    @functools.partial(
        pl.kernel,
        mesh=mesh,
        compiler_params=pltpu.CompilerParams(use_tc_tiling_on_sc=True),
        out_type=jax.ShapeDtypeStruct((total_quads, QD), jnp.float32),
        scratch_types=[
            pltpu.VMEM((per_w,), jnp.int32),
            [pltpu.VMEM((CHUNK, QD), jnp.float32) for _ in range(NBUF)],
            [pltpu.SemaphoreType.DMA for _ in range(NBUF)],
            [pltpu.SemaphoreType.DMA for _ in range(NBUF)],
        ],
    )
    def gather_kernel(table_hbm, idx_hbm, out_hbm, idx_v, rows, gsems, ssems):
        sid = lax.axis_index("s")
        wid = sid * NC + lax.axis_index("c")
        base = wid * per_w
        pltpu.sync_copy(idx_hbm.at[pl.ds(base, per_w)], idx_v)

        def group(g, carry):
            for bf in range(NBUF):
                off = (g * NBUF + bf) * CHUNK

                @pl.when(g > 0)
                def _wait_prev_store():
                    pltpu.make_async_copy(
                        rows[bf], out_hbm.at[pl.ds(base + off, CHUNK)],
                        ssems[bf]).wait()

                pltpu.async_copy(
                    table_hbm.at[idx_v.at[pl.ds(off, CHUNK)]],
                    rows[bf], gsems[bf])
            for bf in range(NBUF):
                off = (g * NBUF + bf) * CHUNK
                pltpu.make_async_copy(
                    table_hbm.at[idx_v.at[pl.ds(off, CHUNK)]],
                    rows[bf], gsems[bf]).wait()
                pltpu.async_copy(rows[bf], out_hbm.at[pl.ds(base + off, CHUNK)],
                                 ssems[bf])
            return carry

        lax.fori_loop(0, n_groups, group, 0)
        for bf in range(NBUF):
            pltpu.make_async_copy(
                rows[bf], out_hbm.at[pl.ds(base, CHUNK)], ssems[bf]).wait()

    return gather_kernel


def kernel(x, pe, gamma, beta, W, b):
    Bb, Ll = x.shape
    total_quads = (Bb * Ll) // 4
    table4, qidx = _prep(pe, gamma, beta, W, b, x.astype(jnp.int32))
    out = _make_gather(total_quads)(table4, qidx.reshape(total_quads))
    return out.reshape(Bb, Ll, EMB_DIM)
